# baseline (speedup 1.0000x reference)
import functools

import jax
import jax.numpy as jnp
from jax import lax
from jax.experimental import pallas as pl
from jax.experimental.pallas import tpu as pltpu
from jax.experimental.pallas import tpu_sc as plsc

B, T, H = 32, 1024, 768
NC, NS, L = 2, 16, 16
NW = NC * NS
G = 32
NCHUNK = T // G
NJ = H // L
CB = 16
EPS = 1e-12


def _rsqrt_newton(v):
    i = lax.bitcast_convert_type(v, jnp.int32)
    i = jnp.full_like(i, 0x5F3759DF) - lax.shift_right_arithmetic(i, jnp.full_like(i, 1))
    y = lax.bitcast_convert_type(i, jnp.float32)
    for _ in range(3):
        y = y * (jnp.float32(1.5) - jnp.float32(0.5) * v * y * y)
    return y


def _lane_perm(x, perm):
    dn = lax.GatherDimensionNumbers(
        offset_dims=(), collapsed_slice_dims=(0,), start_index_map=(0,))
    return lax.gather(x, perm[:, None], dimension_numbers=dn,
                      slice_sizes=(1,),
                      mode=lax.GatherScatterMode.PROMISE_IN_BOUNDS)


def _lane_allsum(x):
    lanes = lax.iota(jnp.int32, L)
    for m in (1, 2, 4, 8):
        x = x + _lane_perm(x, lax.bitwise_xor(lanes, jnp.full_like(lanes, m)))
    return x


def _body(ids_hbm, word_hbm, pos_hbm, gamma_hbm, beta_hbm, out_hbm,
          idxall, wbuf0, wbuf1, pbuf0, pbuf1, obuf, stats, gv, bv,
          gsem0, gsem1, psem0, psem1, osem):
    w = lax.axis_index("s") * NC + lax.axis_index("c")
    wbuf = (wbuf0, wbuf1)
    pbuf = (pbuf0, pbuf1)
    gsem = (gsem0, gsem1)
    psem = (psem0, psem1)
    pltpu.sync_copy(gamma_hbm, gv)
    pltpu.sync_copy(beta_hbm, bv)
    pltpu.sync_copy(ids_hbm.at[w], idxall)

    def start_fetch(ci, b):
        t0 = ci * G
        pltpu.async_copy(word_hbm.at[idxall.at[pl.ds(t0, G)]], wbuf[b], gsem[b])
        pltpu.async_copy(pos_hbm.at[pl.ds(t0, G)], pbuf[b], psem[b])

    def pass1(b):
        xb, pb = wbuf[b], pbuf[b]

        @plsc.parallel_loop(0, G, step=1, unroll=4)
        def row(r):
            acc = [jnp.zeros((L,), jnp.float32) for _ in range(4)]
            qcc = [jnp.zeros((L,), jnp.float32) for _ in range(4)]
            for j in range(0, NJ, 4):
                for u in range(4):
                    x = xb[r, pl.ds((j + u) * L, L)] + pb[r, pl.ds((j + u) * L, L)]
                    xb[r, pl.ds((j + u) * L, L)] = x
                    acc[u] = acc[u] + x
                    qcc[u] = qcc[u] + x * x
            s1 = _lane_allsum((acc[0] + acc[1]) + (acc[2] + acc[3]))
            s2 = _lane_allsum((qcc[0] + qcc[1]) + (qcc[2] + qcc[3]))
            mean = s1 * jnp.float32(1.0 / H)
            var = s2 * jnp.float32(1.0 / H) - mean * mean
            a = _rsqrt_newton(var + jnp.float32(EPS))
            bm = -mean * a
            stats[2 * r] = a[0]
            stats[2 * r + 1] = bm[0]

    def pass2(b):
        xb = wbuf[b]

        @plsc.parallel_loop(0, NJ // CB)
        def colblk(jb):
            gs = [gv[pl.ds((jb * CB + u) * L, L)] for u in range(CB)]
            bts = [bv[pl.ds((jb * CB + u) * L, L)] for u in range(CB)]

            @plsc.parallel_loop(0, G, step=1, unroll=2)
            def rows(r):
                a = stats[2 * r]
                bm = stats[2 * r + 1]
                for u in range(CB):
                    x = xb[r, pl.ds((jb * CB + u) * L, L)]
                    obuf[r, pl.ds((jb * CB + u) * L, L)] = (x * a + bm) * gs[u] + bts[u]

    def out_wait():
        pltpu.make_async_copy(obuf, out_hbm.at[w, pl.ds(0, G)], osem).wait()

    start_fetch(0, 0)

    def pair(ci2, carry):
        for bpar in (0, 1):
            ci = ci2 * 2 + bpar
            pltpu.make_async_copy(word_hbm.at[idxall.at[pl.ds(0, G)]], wbuf[bpar], gsem[bpar]).wait()
            pltpu.make_async_copy(pos_hbm.at[pl.ds(0, G)], pbuf[bpar], psem[bpar]).wait()
            if bpar == 0:
                start_fetch(ci + 1, 1)
            else:
                @pl.when(ci2 < NCHUNK // 2 - 1)
                def _():
                    start_fetch(ci + 1, 0)
            pass1(bpar)
            @pl.when(ci > 0)
            def _():
                out_wait()
            pass2(bpar)
            pltpu.async_copy(obuf, out_hbm.at[w, pl.ds(ci * G, G)], osem)
        return carry

    lax.fori_loop(0, NCHUNK // 2, pair, 0)
    out_wait()


_mesh = plsc.VectorSubcoreMesh(core_axis_name="c", subcore_axis_name="s")

_embed_ln = functools.partial(
    pl.kernel,
    out_type=jax.ShapeDtypeStruct((B, T, H), jnp.float32),
    mesh=_mesh,
    scratch_types=[
        pltpu.VMEM((T,), jnp.int32),
        pltpu.VMEM((G, H), jnp.float32),
        pltpu.VMEM((G, H), jnp.float32),
        pltpu.VMEM((G, H), jnp.float32),
        pltpu.VMEM((G, H), jnp.float32),
        pltpu.VMEM((G, H), jnp.float32),
        pltpu.SMEM((2 * G,), jnp.float32),
        pltpu.VMEM((H,), jnp.float32),
        pltpu.VMEM((H,), jnp.float32),
        pltpu.SemaphoreType.DMA,
        pltpu.SemaphoreType.DMA,
        pltpu.SemaphoreType.DMA,
        pltpu.SemaphoreType.DMA,
        pltpu.SemaphoreType.DMA,
    ],
)(_body)


@jax.jit
def kernel(input_ids, word_emb, pos_emb, ln_gamma, ln_beta):
    return _embed_ln(input_ids.astype(jnp.int32), word_emb, pos_emb,
                     ln_gamma, ln_beta)


# speedup vs baseline: 1.0026x; 1.0026x over previous
import functools

import jax
import jax.numpy as jnp
from jax import lax
from jax.experimental import pallas as pl
from jax.experimental.pallas import tpu as pltpu
from jax.experimental.pallas import tpu_sc as plsc

B, T, H = 32, 1024, 768
NC, NS, L = 2, 16, 16
NW = NC * NS
G = 32
NCHUNK = T // G
NJ = H // L
CB = 8
EPS = 1e-12


def _rsqrt_newton(v):
    i = lax.bitcast_convert_type(v, jnp.int32)
    i = jnp.full_like(i, 0x5F3759DF) - lax.shift_right_arithmetic(i, jnp.full_like(i, 1))
    y = lax.bitcast_convert_type(i, jnp.float32)
    for _ in range(3):
        y = y * (jnp.float32(1.5) - jnp.float32(0.5) * v * y * y)
    return y


def _lane_perm(x, perm):
    dn = lax.GatherDimensionNumbers(
        offset_dims=(), collapsed_slice_dims=(0,), start_index_map=(0,))
    return lax.gather(x, perm[:, None], dimension_numbers=dn,
                      slice_sizes=(1,),
                      mode=lax.GatherScatterMode.PROMISE_IN_BOUNDS)


def _lane_allsum(x):
    lanes = lax.iota(jnp.int32, L)
    for m in (1, 2, 4, 8):
        x = x + _lane_perm(x, lax.bitwise_xor(lanes, jnp.full_like(lanes, m)))
    return x


def _body(ids_hbm, word_hbm, pos_hbm, gamma_hbm, beta_hbm, out_hbm,
          idxall, wbuf0, wbuf1, pbuf0, pbuf1, obuf, stats, gv, bv,
          gsem0, gsem1, psem0, psem1, osem):
    w = lax.axis_index("s") * NC + lax.axis_index("c")
    wbuf = (wbuf0, wbuf1)
    pbuf = (pbuf0, pbuf1)
    gsem = (gsem0, gsem1)
    psem = (psem0, psem1)
    pltpu.sync_copy(ids_hbm.at[w], idxall)

    def start_fetch(ci, b):
        t0 = ci * G
        pltpu.async_copy(word_hbm.at[idxall.at[pl.ds(t0, G)]], wbuf[b], gsem[b])
        pltpu.async_copy(pos_hbm.at[pl.ds(t0, G)], pbuf[b], psem[b])

    def pass1(b):
        xb, pb = wbuf[b], pbuf[b]

        @plsc.parallel_loop(0, G, step=1, unroll=4)
        def row(r):
            acc = [jnp.zeros((L,), jnp.float32) for _ in range(4)]
            qcc = [jnp.zeros((L,), jnp.float32) for _ in range(4)]
            for j in range(0, NJ, 4):
                for u in range(4):
                    x = xb[r, pl.ds((j + u) * L, L)] + pb[r, pl.ds((j + u) * L, L)]
                    xb[r, pl.ds((j + u) * L, L)] = x
                    acc[u] = acc[u] + x
                    qcc[u] = qcc[u] + x * x
            s1 = _lane_allsum((acc[0] + acc[1]) + (acc[2] + acc[3]))
            s2 = _lane_allsum((qcc[0] + qcc[1]) + (qcc[2] + qcc[3]))
            mean = s1 * jnp.float32(1.0 / H)
            var = s2 * jnp.float32(1.0 / H) - mean * mean
            a = _rsqrt_newton(var + jnp.float32(EPS))
            bm = -mean * a
            stats[2 * r] = a[0]
            stats[2 * r + 1] = bm[0]

    def pass2(b):
        xb = wbuf[b]

        @plsc.parallel_loop(0, NJ // CB)
        def colblk(jb):
            gs = [gv[pl.ds((jb * CB + u) * L, L)] for u in range(CB)]
            bts = [bv[pl.ds((jb * CB + u) * L, L)] for u in range(CB)]

            @plsc.parallel_loop(0, G, step=1, unroll=2)
            def rows(r):
                a = stats[2 * r]
                bm = stats[2 * r + 1]
                for u in range(CB):
                    x = xb[r, pl.ds((jb * CB + u) * L, L)]
                    obuf[r, pl.ds((jb * CB + u) * L, L)] = (x * a + bm) * gs[u] + bts[u]

    def out_wait():
        pltpu.make_async_copy(obuf, out_hbm.at[w, pl.ds(0, G)], osem).wait()

    start_fetch(0, 0)
    pltpu.sync_copy(gamma_hbm, gv)
    pltpu.sync_copy(beta_hbm, bv)

    def pair(ci2, carry):
        for bpar in (0, 1):
            ci = ci2 * 2 + bpar
            pltpu.make_async_copy(word_hbm.at[idxall.at[pl.ds(0, G)]], wbuf[bpar], gsem[bpar]).wait()
            pltpu.make_async_copy(pos_hbm.at[pl.ds(0, G)], pbuf[bpar], psem[bpar]).wait()
            if bpar == 0:
                start_fetch(ci + 1, 1)
            else:
                @pl.when(ci2 < NCHUNK // 2 - 1)
                def _():
                    start_fetch(ci + 1, 0)
            pass1(bpar)
            @pl.when(ci > 0)
            def _():
                out_wait()
            pass2(bpar)
            pltpu.async_copy(obuf, out_hbm.at[w, pl.ds(ci * G, G)], osem)
        return carry

    lax.fori_loop(0, NCHUNK // 2, pair, 0)
    out_wait()


_mesh = plsc.VectorSubcoreMesh(core_axis_name="c", subcore_axis_name="s")

_embed_ln = functools.partial(
    pl.kernel,
    out_type=jax.ShapeDtypeStruct((B, T, H), jnp.float32),
    mesh=_mesh,
    scratch_types=[
        pltpu.VMEM((T,), jnp.int32),
        pltpu.VMEM((G, H), jnp.float32),
        pltpu.VMEM((G, H), jnp.float32),
        pltpu.VMEM((G, H), jnp.float32),
        pltpu.VMEM((G, H), jnp.float32),
        pltpu.VMEM((G, H), jnp.float32),
        pltpu.SMEM((2 * G,), jnp.float32),
        pltpu.VMEM((H,), jnp.float32),
        pltpu.VMEM((H,), jnp.float32),
        pltpu.SemaphoreType.DMA,
        pltpu.SemaphoreType.DMA,
        pltpu.SemaphoreType.DMA,
        pltpu.SemaphoreType.DMA,
        pltpu.SemaphoreType.DMA,
    ],
)(_body)


@jax.jit
def kernel(input_ids, word_emb, pos_emb, ln_gamma, ln_beta):
    return _embed_ln(input_ids.astype(jnp.int32), word_emb, pos_emb,
                     ln_gamma, ln_beta)


# speedup vs baseline: 1.0075x; 1.0049x over previous
import functools

import jax
import jax.numpy as jnp
from jax import lax
from jax.experimental import pallas as pl
from jax.experimental.pallas import tpu as pltpu
from jax.experimental.pallas import tpu_sc as plsc

B, T, H = 32, 1024, 768
NC, NS, L = 2, 16, 16
NW = NC * NS
G = 32
NCHUNK = T // G
NJ = H // L
CB = 8
EPS = 1e-12


def _rsqrt_newton(v):
    i = lax.bitcast_convert_type(v, jnp.int32)
    i = jnp.full_like(i, 0x5F3759DF) - lax.shift_right_arithmetic(i, jnp.full_like(i, 1))
    y = lax.bitcast_convert_type(i, jnp.float32)
    for _ in range(3):
        y = y * (jnp.float32(1.5) - jnp.float32(0.5) * v * y * y)
    return y


def _lane_perm(x, perm):
    dn = lax.GatherDimensionNumbers(
        offset_dims=(), collapsed_slice_dims=(0,), start_index_map=(0,))
    return lax.gather(x, perm[:, None], dimension_numbers=dn,
                      slice_sizes=(1,),
                      mode=lax.GatherScatterMode.PROMISE_IN_BOUNDS)


def _lane_allsum(x):
    lanes = lax.iota(jnp.int32, L)
    for m in (1, 2, 4, 8):
        x = x + _lane_perm(x, lax.bitwise_xor(lanes, jnp.full_like(lanes, m)))
    return x


def _body(ids_hbm, word_hbm, pos_hbm, gamma_hbm, beta_hbm, out_hbm,
          idxall, wbuf0, wbuf1, pbuf0, pbuf1, obuf, stats, gv, bv,
          gsem0, gsem1, psem0, psem1, osem):
    w = lax.axis_index("s") * NC + lax.axis_index("c")
    wbuf = (wbuf0, wbuf1)
    pbuf = (pbuf0, pbuf1)
    gsem = (gsem0, gsem1)
    psem = (psem0, psem1)
    pltpu.sync_copy(ids_hbm.at[w], idxall)

    def start_fetch(ci, b):
        t0 = ci * G
        pltpu.async_copy(word_hbm.at[idxall.at[pl.ds(t0, G)]], wbuf[b], gsem[b])
        pltpu.async_copy(pos_hbm.at[pl.ds(t0, G)], pbuf[b], psem[b])

    def pass1(b):
        xb, pb = wbuf[b], pbuf[b]

        @plsc.parallel_loop(0, G, step=1, unroll=4)
        def row(r):
            acc = [jnp.zeros((L,), jnp.float32) for _ in range(4)]
            qcc = [jnp.zeros((L,), jnp.float32) for _ in range(4)]
            for j in range(0, NJ, 4):
                for u in range(4):
                    x = xb[r, pl.ds((j + u) * L, L)] + pb[r, pl.ds((j + u) * L, L)]
                    xb[r, pl.ds((j + u) * L, L)] = x
                    acc[u] = acc[u] + x
                    qcc[u] = qcc[u] + x * x
            s1 = _lane_allsum((acc[0] + acc[1]) + (acc[2] + acc[3]))
            s2 = _lane_allsum((qcc[0] + qcc[1]) + (qcc[2] + qcc[3]))
            mean = s1 * jnp.float32(1.0 / H)
            var = s2 * jnp.float32(1.0 / H) - mean * mean
            a = _rsqrt_newton(var + jnp.float32(EPS))
            bm = -mean * a
            stats[2 * r] = a[0]
            stats[2 * r + 1] = bm[0]

    def pass2(b):
        xb = wbuf[b]

        @plsc.parallel_loop(0, NJ // CB)
        def colblk(jb):
            gs = [gv[pl.ds((jb * CB + u) * L, L)] for u in range(CB)]
            bts = [bv[pl.ds((jb * CB + u) * L, L)] for u in range(CB)]

            @plsc.parallel_loop(0, G, step=1, unroll=4)
            def rows(r):
                a = stats[2 * r]
                bm = stats[2 * r + 1]
                for u in range(CB):
                    x = xb[r, pl.ds((jb * CB + u) * L, L)]
                    obuf[r, pl.ds((jb * CB + u) * L, L)] = (x * a + bm) * gs[u] + bts[u]

    def out_wait():
        pltpu.make_async_copy(obuf, out_hbm.at[w, pl.ds(0, G)], osem).wait()

    start_fetch(0, 0)
    pltpu.sync_copy(gamma_hbm, gv)
    pltpu.sync_copy(beta_hbm, bv)

    def pair(ci2, carry):
        for bpar in (0, 1):
            ci = ci2 * 2 + bpar
            pltpu.make_async_copy(word_hbm.at[idxall.at[pl.ds(0, G)]], wbuf[bpar], gsem[bpar]).wait()
            pltpu.make_async_copy(pos_hbm.at[pl.ds(0, G)], pbuf[bpar], psem[bpar]).wait()
            if bpar == 0:
                start_fetch(ci + 1, 1)
            else:
                @pl.when(ci2 < NCHUNK // 2 - 1)
                def _():
                    start_fetch(ci + 1, 0)
            pass1(bpar)
            @pl.when(ci > 0)
            def _():
                out_wait()
            pass2(bpar)
            pltpu.async_copy(obuf, out_hbm.at[w, pl.ds(ci * G, G)], osem)
        return carry

    lax.fori_loop(0, NCHUNK // 2, pair, 0)
    out_wait()


_mesh = plsc.VectorSubcoreMesh(core_axis_name="c", subcore_axis_name="s")

_embed_ln = functools.partial(
    pl.kernel,
    out_type=jax.ShapeDtypeStruct((B, T, H), jnp.float32),
    mesh=_mesh,
    scratch_types=[
        pltpu.VMEM((T,), jnp.int32),
        pltpu.VMEM((G, H), jnp.float32),
        pltpu.VMEM((G, H), jnp.float32),
        pltpu.VMEM((G, H), jnp.float32),
        pltpu.VMEM((G, H), jnp.float32),
        pltpu.VMEM((G, H), jnp.float32),
        pltpu.SMEM((2 * G,), jnp.float32),
        pltpu.VMEM((H,), jnp.float32),
        pltpu.VMEM((H,), jnp.float32),
        pltpu.SemaphoreType.DMA,
        pltpu.SemaphoreType.DMA,
        pltpu.SemaphoreType.DMA,
        pltpu.SemaphoreType.DMA,
        pltpu.SemaphoreType.DMA,
    ],
)(_body)


@jax.jit
def kernel(input_ids, word_emb, pos_emb, ln_gamma, ln_beta):
    return _embed_ln(input_ids.astype(jnp.int32), word_emb, pos_emb,
                     ln_gamma, ln_beta)


# speedup vs baseline: 1.0848x; 1.0766x over previous
import functools

import jax
import jax.numpy as jnp
from jax import lax
from jax.experimental import pallas as pl
from jax.experimental.pallas import tpu as pltpu
from jax.experimental.pallas import tpu_sc as plsc

B, T, H = 32, 1024, 768
NC, NS, L = 2, 16, 16
NW = NC * NS
G = 16
NCHUNK = T // G
NJ = H // L
CB = 8
EPS = 1e-12


def _rsqrt_newton(v):
    i = lax.bitcast_convert_type(v, jnp.int32)
    i = jnp.full_like(i, 0x5F3759DF) - lax.shift_right_arithmetic(i, jnp.full_like(i, 1))
    y = lax.bitcast_convert_type(i, jnp.float32)
    for _ in range(3):
        y = y * (jnp.float32(1.5) - jnp.float32(0.5) * v * y * y)
    return y


def _lane_perm(x, perm):
    dn = lax.GatherDimensionNumbers(
        offset_dims=(), collapsed_slice_dims=(0,), start_index_map=(0,))
    return lax.gather(x, perm[:, None], dimension_numbers=dn,
                      slice_sizes=(1,),
                      mode=lax.GatherScatterMode.PROMISE_IN_BOUNDS)


def _lane_allsum(x):
    lanes = lax.iota(jnp.int32, L)
    for m in (1, 2, 4, 8):
        x = x + _lane_perm(x, lax.bitwise_xor(lanes, jnp.full_like(lanes, m)))
    return x


def _body(ids_hbm, word_hbm, pos_hbm, gamma_hbm, beta_hbm, out_hbm,
          idxall, wbuf0, wbuf1, pbuf0, pbuf1, obuf, pos_sh, stats, gv, bv,
          gsem0, gsem1, psem0, psem1, osem):
    cid = lax.axis_index("c")
    sid = lax.axis_index("s")
    w = sid * NC + cid
    wbuf = (wbuf0, wbuf1)
    pbuf = (pbuf0, pbuf1)
    gsem = (gsem0, gsem1)
    psem = (psem0, psem1)
    pltpu.sync_copy(ids_hbm.at[w], idxall)
    rpt = T // NS
    pltpu.sync_copy(pos_hbm.at[pl.ds(sid * rpt, rpt)],
                    pos_sh.at[pl.ds(sid * rpt, rpt)])
    plsc.subcore_barrier()

    def start_fetch(ci, b):
        t0 = ci * G
        pltpu.async_copy(word_hbm.at[idxall.at[pl.ds(t0, G)]], wbuf[b], gsem[b])
        pltpu.async_copy(pos_sh.at[pl.ds(t0, G)], pbuf[b], psem[b])

    def pass1(b):
        xb, pb = wbuf[b], pbuf[b]

        @plsc.parallel_loop(0, G, step=1, unroll=4)
        def row(r):
            acc = [jnp.zeros((L,), jnp.float32) for _ in range(4)]
            qcc = [jnp.zeros((L,), jnp.float32) for _ in range(4)]
            for j in range(0, NJ, 4):
                for u in range(4):
                    x = xb[r, pl.ds((j + u) * L, L)] + pb[r, pl.ds((j + u) * L, L)]
                    xb[r, pl.ds((j + u) * L, L)] = x
                    acc[u] = acc[u] + x
                    qcc[u] = qcc[u] + x * x
            s1 = _lane_allsum((acc[0] + acc[1]) + (acc[2] + acc[3]))
            s2 = _lane_allsum((qcc[0] + qcc[1]) + (qcc[2] + qcc[3]))
            mean = s1 * jnp.float32(1.0 / H)
            var = s2 * jnp.float32(1.0 / H) - mean * mean
            a = _rsqrt_newton(var + jnp.float32(EPS))
            bm = -mean * a
            stats[2 * r] = a[0]
            stats[2 * r + 1] = bm[0]

    def pass2(b):
        xb = wbuf[b]

        @plsc.parallel_loop(0, NJ // CB)
        def colblk(jb):
            gs = [gv[pl.ds((jb * CB + u) * L, L)] for u in range(CB)]
            bts = [bv[pl.ds((jb * CB + u) * L, L)] for u in range(CB)]

            @plsc.parallel_loop(0, G, step=1, unroll=4)
            def rows(r):
                a = stats[2 * r]
                bm = stats[2 * r + 1]
                for u in range(CB):
                    x = xb[r, pl.ds((jb * CB + u) * L, L)]
                    obuf[r, pl.ds((jb * CB + u) * L, L)] = (x * a + bm) * gs[u] + bts[u]

    def out_wait():
        pltpu.make_async_copy(obuf, out_hbm.at[w, pl.ds(0, G)], osem).wait()

    start_fetch(0, 0)
    pltpu.sync_copy(gamma_hbm, gv)
    pltpu.sync_copy(beta_hbm, bv)

    def pair(ci2, carry):
        for bpar in (0, 1):
            ci = ci2 * 2 + bpar
            pltpu.make_async_copy(word_hbm.at[idxall.at[pl.ds(0, G)]], wbuf[bpar], gsem[bpar]).wait()
            pltpu.make_async_copy(pos_sh.at[pl.ds(0, G)], pbuf[bpar], psem[bpar]).wait()
            if bpar == 0:
                start_fetch(ci + 1, 1)
            else:
                @pl.when(ci2 < NCHUNK // 2 - 1)
                def _():
                    start_fetch(ci + 1, 0)
            pass1(bpar)
            @pl.when(ci > 0)
            def _():
                out_wait()
            pass2(bpar)
            pltpu.async_copy(obuf, out_hbm.at[w, pl.ds(ci * G, G)], osem)
        return carry

    lax.fori_loop(0, NCHUNK // 2, pair, 0)
    out_wait()


_mesh = plsc.VectorSubcoreMesh(core_axis_name="c", subcore_axis_name="s")

_embed_ln = functools.partial(
    pl.kernel,
    out_type=jax.ShapeDtypeStruct((B, T, H), jnp.float32),
    mesh=_mesh,
    scratch_types=[
        pltpu.VMEM((T,), jnp.int32),
        pltpu.VMEM((G, H), jnp.float32),
        pltpu.VMEM((G, H), jnp.float32),
        pltpu.VMEM((G, H), jnp.float32),
        pltpu.VMEM((G, H), jnp.float32),
        pltpu.VMEM((G, H), jnp.float32),
        pltpu.VMEM_SHARED((T, H), jnp.float32),
        pltpu.SMEM((2 * G,), jnp.float32),
        pltpu.VMEM((H,), jnp.float32),
        pltpu.VMEM((H,), jnp.float32),
        pltpu.SemaphoreType.DMA,
        pltpu.SemaphoreType.DMA,
        pltpu.SemaphoreType.DMA,
        pltpu.SemaphoreType.DMA,
        pltpu.SemaphoreType.DMA,
    ],
)(_body)


@jax.jit
def kernel(input_ids, word_emb, pos_emb, ln_gamma, ln_beta):
    return _embed_ln(input_ids.astype(jnp.int32), word_emb, pos_emb,
                     ln_gamma, ln_beta)
